# Initial kernel scaffold; baseline (speedup 1.0000x reference)
import jax
import jax.numpy as jnp
from jax.experimental import pallas as pl


def kernel(con_output, proj_W, proj_b, in_proj_W, in_proj_b, out_proj_W, out_proj_b, ln_gamma, ln_beta, gate_W, gate_b, eW1, eb1, eW2, eb2, eW3, eb3, eW4, eb4, eW5, eb5):
    raise NotImplementedError("write your pallas kernel here")


# speedup vs baseline: 1.4692x; 1.4692x over previous
import functools

import jax
import jax.numpy as jnp
import numpy as np
from jax import lax
from jax.experimental import pallas as pl
from jax.experimental.pallas import tpu as pltpu
from jax.experimental.pallas import tpu_sc as plsc

B = 512
D = 1024
NH = 8
DH = D // NH
E = 16
K = 2
P = B * K
R = 64
CAP = 2048
NBLK = CAP // R
SC_NC = 2
SC_NS = 16
NW = SC_NC * SC_NS
BPW = CAP // NW


def _dott(a, b):
    return lax.dot_general(a, b, (((1,), (1,)), ((), ())),
                           preferred_element_type=jnp.float32)


def _dot(a, b):
    return lax.dot_general(a, b, (((1,), (0,)), ((), ())),
                           preferred_element_type=jnp.float32)


def _gelu(x):
    return 0.5 * x * (1.0 + lax.erf(x * np.float32(1.0 / np.sqrt(2.0))))


def _qkv_body(x_ref, pw_ref, pb_ref, iw_ref, ib_ref, qkv_ref):
    proj = _dott(x_ref[...], pw_ref[...]) + pb_ref[...]
    qkv_ref[...] = _dott(proj, iw_ref[...]) + ib_ref[...]


def _qkv_call(x, proj_W, proj_b, in_proj_W, in_proj_b):
    return pl.pallas_call(
        _qkv_body,
        out_shape=jax.ShapeDtypeStruct((B, 3 * D), jnp.float32),
    )(x, proj_W, proj_b.reshape(1, D), in_proj_W, in_proj_b.reshape(1, 3 * D))


def _attn_body(q_ref, k_ref, v_ref, o_ref):
    s = _dott(q_ref[...], k_ref[...]) * np.float32(1.0 / np.sqrt(DH))
    m = jnp.max(s, axis=-1, keepdims=True)
    e = jnp.exp(s - m)
    attn = e / jnp.sum(e, axis=-1, keepdims=True)
    o_ref[...] = _dot(attn, v_ref[...])


def _attn_call(qkv):
    return pl.pallas_call(
        _attn_body,
        grid=(NH,),
        in_specs=[
            pl.BlockSpec((B, DH), lambda h: (0, h)),
            pl.BlockSpec((B, DH), lambda h: (0, NH + h)),
            pl.BlockSpec((B, DH), lambda h: (0, 2 * NH + h)),
        ],
        out_specs=pl.BlockSpec((B, DH), lambda h: (0, h)),
        out_shape=jax.ShapeDtypeStruct((B, D), jnp.float32),
    )(qkv, qkv, qkv)


def _route_body(o_ref, ow_ref, ob_ref, lg_ref, lb_ref, gw_ref, gb_ref,
                z_ref, disp_ref, bexp_ref, bact_ref, pos_ref, ef_ref, wts_ref):
    ao = _dott(o_ref[...], ow_ref[...]) + ob_ref[...]
    mu = jnp.mean(ao, axis=-1, keepdims=True)
    var = jnp.mean((ao - mu) ** 2, axis=-1, keepdims=True)
    z = (ao - mu) / jnp.sqrt(var + np.float32(1e-5)) * lg_ref[...] + lb_ref[...]
    z_ref[...] = z

    g = _dott(z, gw_ref[...]) + gb_ref[...]
    ioL = lax.broadcasted_iota(jnp.int32, (B, E), 1)
    l1 = jnp.max(g, axis=1, keepdims=True)
    i1 = jnp.min(jnp.where(g == l1, ioL, E), axis=1, keepdims=True)
    gm = jnp.where(ioL == i1, -jnp.inf, g)
    l2 = jnp.max(gm, axis=1, keepdims=True)
    i2 = jnp.min(jnp.where(gm == l2, ioL, E), axis=1, keepdims=True)
    w1 = 1.0 / (1.0 + jnp.exp(l2 - l1))
    wts_ref[...] = jnp.concatenate([w1, 1.0 - w1], axis=1)
    ef_ref[...] = jnp.concatenate([i1, i2], axis=0)

    oh = jnp.concatenate([ioL == i1, ioL == i2], axis=0).astype(jnp.float32)
    counts = jnp.sum(oh, axis=0, keepdims=True).astype(jnp.int32)
    padded = ((counts + (R - 1)) // R) * R
    mr = lax.broadcasted_iota(jnp.int32, (E, E), 0)
    mc = lax.broadcasted_iota(jnp.int32, (E, E), 1)
    mstrict = (mr < mc).astype(jnp.float32)
    off_row = _dot(padded.astype(jnp.float32), mstrict)
    tr = lax.broadcasted_iota(jnp.int32, (P, P), 0)
    tc = lax.broadcasted_iota(jnp.int32, (P, P), 1)
    tstrict = (tc < tr).astype(jnp.float32)
    rankmat = _dot(tstrict, oh)
    rank = jnp.sum(rankmat * oh, axis=1, keepdims=True)
    posoff = jnp.sum(oh * off_row, axis=1, keepdims=True)
    pos = (posoff + rank).astype(jnp.int32)
    pos_ref[...] = pos

    pos_oh = (lax.broadcasted_iota(jnp.int32, (P, CAP), 1) == pos)
    tokf = (lax.broadcasted_iota(jnp.int32, (1, P), 1) % B).astype(jnp.float32)
    disp_ref[...] = _dot(tokf, pos_oh.astype(jnp.float32)).astype(jnp.int32)

    bs = lax.broadcasted_iota(jnp.int32, (NBLK, 1), 0) * R
    offi = off_row.astype(jnp.int32)
    bexp_ref[...] = jnp.sum((offi <= bs).astype(jnp.int32), axis=1,
                            keepdims=True) - 1
    total = jnp.sum(padded)
    bact_ref[...] = (bs < total).astype(jnp.int32)


def _route_call(o, out_proj_W, out_proj_b, ln_gamma, ln_beta, gate_W, gate_b):
    return pl.pallas_call(
        _route_body,
        out_shape=(
            jax.ShapeDtypeStruct((B, D), jnp.float32),
            jax.ShapeDtypeStruct((1, CAP), jnp.int32),
            jax.ShapeDtypeStruct((NBLK, 1), jnp.int32),
            jax.ShapeDtypeStruct((NBLK, 1), jnp.int32),
            jax.ShapeDtypeStruct((P, 1), jnp.int32),
            jax.ShapeDtypeStruct((P, 1), jnp.int32),
            jax.ShapeDtypeStruct((B, K), jnp.float32),
        ),
    )(o, out_proj_W, out_proj_b.reshape(1, D), ln_gamma.reshape(1, D),
      ln_beta.reshape(1, D), gate_W, gate_b.reshape(1, E))


def _sc_gather_body(z_hbm, idx_hbm, out_hbm, idx_v, rows_v, sem):
    wid = lax.axis_index("s") * SC_NC + lax.axis_index("c")
    base = wid * BPW
    pltpu.sync_copy(idx_hbm.at[pl.ds(base, BPW)], idx_v)
    pltpu.async_copy(z_hbm.at[idx_v], rows_v, sem).wait()
    pltpu.sync_copy(rows_v, out_hbm.at[pl.ds(base, BPW)])


def _sc_gather(z, disp):
    mesh = plsc.VectorSubcoreMesh(core_axis_name="c", subcore_axis_name="s")
    return pl.kernel(
        _sc_gather_body,
        out_type=jax.ShapeDtypeStruct((CAP, D), jnp.float32),
        mesh=mesh,
        scratch_types=[
            pltpu.VMEM((BPW,), jnp.int32),
            pltpu.VMEM((BPW, D), jnp.float32),
            pltpu.SemaphoreType.DMA,
        ],
    )(z, disp)


def _expert_body(be_ref, act_ref, zd_ref, w1_ref, b1_ref, w2_ref, b2_ref,
                 w3_ref, b3_ref, w4_ref, b4_ref, w5_ref, out_ref):
    j = pl.program_id(0)

    @pl.when(act_ref[j] != 0)
    def _active():
        h = _gelu(_dott(zd_ref[...], w1_ref[0]) + b1_ref[0])
        h = _gelu(_dott(h, w2_ref[0]) + b2_ref[0])
        h = _gelu(_dott(h, w3_ref[0]) + b3_ref[0])
        h = _gelu(_dott(h, w4_ref[0]) + b4_ref[0])
        out_ref[...] = jnp.sum(h * w5_ref[0], axis=1, keepdims=True)

    @pl.when(act_ref[j] == 0)
    def _inactive():
        out_ref[...] = jnp.zeros((R, 1), jnp.float32)


def _expert_call(bexp, bact, zd, eW1, eb1, eW2, eb2, eW3, eb3, eW4, eb4, eW5):
    def _wmap(j, be, act):
        return (be[j], 0, 0)

    grid_spec = pltpu.PrefetchScalarGridSpec(
        num_scalar_prefetch=2,
        grid=(NBLK,),
        in_specs=[
            pl.BlockSpec((R, D), lambda j, be, act: (j, 0)),
            pl.BlockSpec((1, 1024, 1024), _wmap),
            pl.BlockSpec((1, 1, 1024), _wmap),
            pl.BlockSpec((1, 512, 1024), _wmap),
            pl.BlockSpec((1, 1, 512), _wmap),
            pl.BlockSpec((1, 256, 512), _wmap),
            pl.BlockSpec((1, 1, 256), _wmap),
            pl.BlockSpec((1, 128, 256), _wmap),
            pl.BlockSpec((1, 1, 128), _wmap),
            pl.BlockSpec((1, 1, 128), _wmap),
        ],
        out_specs=pl.BlockSpec((R, 1), lambda j, be, act: (j, 0)),
    )
    return pl.pallas_call(
        _expert_body,
        grid_spec=grid_spec,
        out_shape=jax.ShapeDtypeStruct((CAP, 1), jnp.float32),
    )(bexp, bact, zd,
      eW1, eb1.reshape(E, 1, 1024), eW2, eb2.reshape(E, 1, 512),
      eW3, eb3.reshape(E, 1, 256), eW4, eb4.reshape(E, 1, 128), eW5)


def _combine_body(op_ref, pos_ref, ef_ref, wts_ref, eb5_ref, out_ref):
    sel_oh = (lax.broadcasted_iota(jnp.int32, (P, CAP), 1) == pos_ref[...])
    sel = _dot(sel_oh.astype(jnp.float32), op_ref[...])
    eb_oh = (lax.broadcasted_iota(jnp.int32, (P, E), 1) == ef_ref[...])
    s = sel + _dot(eb_oh.astype(jnp.float32), eb5_ref[...])
    w = wts_ref[...]
    out_ref[...] = jax.nn.sigmoid(s[:B, :] * w[:, 0:1] + s[B:, :] * w[:, 1:2])


def _combine_call(op, pos, ef, wts, eb5):
    return pl.pallas_call(
        _combine_body,
        out_shape=jax.ShapeDtypeStruct((B, 1), jnp.float32),
    )(op, pos, ef, wts, eb5)


def kernel(con_output, proj_W, proj_b, in_proj_W, in_proj_b, out_proj_W,
           out_proj_b, ln_gamma, ln_beta, gate_W, gate_b,
           eW1, eb1, eW2, eb2, eW3, eb3, eW4, eb4, eW5, eb5):
    qkv = _qkv_call(con_output, proj_W, proj_b, in_proj_W, in_proj_b)
    o = _attn_call(qkv)
    z, disp, bexp, bact, pos, ef, wts = _route_call(
        o, out_proj_W, out_proj_b, ln_gamma, ln_beta, gate_W, gate_b)
    zd = _sc_gather(z, disp.reshape(CAP))
    op = _expert_call(bexp.reshape(NBLK), bact.reshape(NBLK), zd,
                      eW1, eb1, eW2, eb2, eW3, eb3, eW4, eb4, eW5)
    out = _combine_call(op, pos, ef, wts, eb5)
    return out.reshape(B)


# speedup vs baseline: 1.9094x; 1.2997x over previous
import functools

import jax
import jax.numpy as jnp
import numpy as np
from jax import lax
from jax.experimental import pallas as pl
from jax.experimental.pallas import tpu as pltpu
from jax.experimental.pallas import tpu_sc as plsc

B = 512
D = 1024
NH = 8
DH = D // NH
E = 16
K = 2
P = B * K
R = 64
CAP = 2048
NBLK = CAP // R
SC_NC = 2
SC_NS = 16
NW = SC_NC * SC_NS
BPW = CAP // NW


def _dott(a, b):
    return lax.dot_general(a, b, (((1,), (1,)), ((), ())),
                           preferred_element_type=jnp.float32)


def _dot(a, b):
    return lax.dot_general(a, b, (((1,), (0,)), ((), ())),
                           preferred_element_type=jnp.float32)


def _gelu(x):
    return 0.5 * x * (1.0 + lax.erf(x * np.float32(1.0 / np.sqrt(2.0))))


def _qkv_body(x_ref, pw_ref, pb_ref, iw_ref, ib_ref, qkv_ref):
    proj = _dott(x_ref[...], pw_ref[...]) + pb_ref[...]
    qkv_ref[...] = _dott(proj, iw_ref[...]) + ib_ref[...]


def _qkv_call(x, proj_W, proj_b, in_proj_W, in_proj_b):
    return pl.pallas_call(
        _qkv_body,
        out_shape=jax.ShapeDtypeStruct((B, 3 * D), jnp.float32),
    )(x, proj_W, proj_b.reshape(1, D), in_proj_W, in_proj_b.reshape(1, 3 * D))


def _attn_body(q_ref, k_ref, v_ref, o_ref):
    s = _dott(q_ref[...], k_ref[...]) * np.float32(1.0 / np.sqrt(DH))
    m = jnp.max(s, axis=-1, keepdims=True)
    e = jnp.exp(s - m)
    attn = e / jnp.sum(e, axis=-1, keepdims=True)
    o_ref[...] = _dot(attn, v_ref[...])


def _attn_call(qkv):
    return pl.pallas_call(
        _attn_body,
        grid=(NH,),
        in_specs=[
            pl.BlockSpec((B, DH), lambda h: (0, h)),
            pl.BlockSpec((B, DH), lambda h: (0, NH + h)),
            pl.BlockSpec((B, DH), lambda h: (0, 2 * NH + h)),
        ],
        out_specs=pl.BlockSpec((B, DH), lambda h: (0, h)),
        out_shape=jax.ShapeDtypeStruct((B, D), jnp.float32),
    )(qkv, qkv, qkv)


def _route_body(o_ref, ow_ref, ob_ref, lg_ref, lb_ref, gw_ref, gb_ref,
                z_ref, disp_ref, bexp_ref, bact_ref, pos_ref, ef_ref, wts_ref):
    ao = _dott(o_ref[...], ow_ref[...]) + ob_ref[...]
    mu = jnp.mean(ao, axis=-1, keepdims=True)
    var = jnp.mean((ao - mu) ** 2, axis=-1, keepdims=True)
    z = (ao - mu) / jnp.sqrt(var + np.float32(1e-5)) * lg_ref[...] + lb_ref[...]
    z_ref[...] = z

    g = _dott(z, gw_ref[...]) + gb_ref[...]
    ioL = lax.broadcasted_iota(jnp.int32, (B, E), 1)
    l1 = jnp.max(g, axis=1, keepdims=True)
    i1 = jnp.min(jnp.where(g == l1, ioL, E), axis=1, keepdims=True)
    gm = jnp.where(ioL == i1, -jnp.inf, g)
    l2 = jnp.max(gm, axis=1, keepdims=True)
    i2 = jnp.min(jnp.where(gm == l2, ioL, E), axis=1, keepdims=True)
    w1 = 1.0 / (1.0 + jnp.exp(l2 - l1))
    wts_ref[...] = jnp.concatenate([w1, 1.0 - w1], axis=1)
    ef_ref[...] = jnp.concatenate([i1, i2], axis=0)

    oh = jnp.concatenate([ioL == i1, ioL == i2], axis=0).astype(jnp.float32)
    counts = jnp.sum(oh, axis=0, keepdims=True).astype(jnp.int32)
    padded = ((counts + (R - 1)) // R) * R
    mr = lax.broadcasted_iota(jnp.int32, (E, E), 0)
    mc = lax.broadcasted_iota(jnp.int32, (E, E), 1)
    mstrict = (mr < mc).astype(jnp.float32)
    off_row = _dot(padded.astype(jnp.float32), mstrict)
    tr = lax.broadcasted_iota(jnp.int32, (P, P), 0)
    tc = lax.broadcasted_iota(jnp.int32, (P, P), 1)
    tstrict = (tc < tr).astype(jnp.float32)
    rankmat = _dot(tstrict, oh)
    rank = jnp.sum(rankmat * oh, axis=1, keepdims=True)
    posoff = jnp.sum(oh * off_row, axis=1, keepdims=True)
    pos = (posoff + rank).astype(jnp.int32)
    pos_ref[...] = pos

    pos_oh = (lax.broadcasted_iota(jnp.int32, (P, CAP), 1) == pos)
    tokf = (lax.broadcasted_iota(jnp.int32, (1, P), 1) % B).astype(jnp.float32)
    disp_ref[...] = _dot(tokf, pos_oh.astype(jnp.float32)).astype(jnp.int32)

    bs = lax.broadcasted_iota(jnp.int32, (NBLK, 1), 0) * R
    offi = off_row.astype(jnp.int32)
    bexp_ref[...] = jnp.sum((offi <= bs).astype(jnp.int32), axis=1,
                            keepdims=True) - 1
    total = jnp.sum(padded)
    bact_ref[...] = (bs < total).astype(jnp.int32)


def _route_call(o, out_proj_W, out_proj_b, ln_gamma, ln_beta, gate_W, gate_b):
    return pl.pallas_call(
        _route_body,
        out_shape=(
            jax.ShapeDtypeStruct((B, D), jnp.float32),
            jax.ShapeDtypeStruct((1, CAP), jnp.int32),
            jax.ShapeDtypeStruct((NBLK, 1), jnp.int32),
            jax.ShapeDtypeStruct((NBLK, 1), jnp.int32),
            jax.ShapeDtypeStruct((P, 1), jnp.int32),
            jax.ShapeDtypeStruct((P, 1), jnp.int32),
            jax.ShapeDtypeStruct((B, K), jnp.float32),
        ),
    )(o, out_proj_W, out_proj_b.reshape(1, D), ln_gamma.reshape(1, D),
      ln_beta.reshape(1, D), gate_W, gate_b.reshape(1, E))


TPW = B // NW


def _sc_combine_body(op_hbm, pos_hbm, ef_hbm, w1_hbm, w2_hbm, eb5_hbm,
                     out_hbm, op_v, pos_v, ef_v, w_v, eb5_v, out_v):
    wid = lax.axis_index("s") * SC_NC + lax.axis_index("c")
    base = wid * TPW
    pltpu.sync_copy(op_hbm, op_v)
    pltpu.sync_copy(pos_hbm.at[pl.ds(base, TPW)], pos_v.at[pl.ds(0, TPW)])
    pltpu.sync_copy(pos_hbm.at[pl.ds(B + base, TPW)], pos_v.at[pl.ds(TPW, TPW)])
    pltpu.sync_copy(ef_hbm.at[pl.ds(base, TPW)], ef_v.at[pl.ds(0, TPW)])
    pltpu.sync_copy(ef_hbm.at[pl.ds(B + base, TPW)], ef_v.at[pl.ds(TPW, TPW)])
    pltpu.sync_copy(w1_hbm.at[pl.ds(base, TPW)], w_v.at[pl.ds(0, TPW)])
    pltpu.sync_copy(w2_hbm.at[pl.ds(base, TPW)], w_v.at[pl.ds(TPW, TPW)])
    pltpu.sync_copy(eb5_hbm, eb5_v)
    s1 = plsc.load_gather(op_v, [pos_v[pl.ds(0, TPW)]])
    s2 = plsc.load_gather(op_v, [pos_v[pl.ds(TPW, TPW)]])
    b1 = plsc.load_gather(eb5_v, [ef_v[pl.ds(0, TPW)]])
    b2 = plsc.load_gather(eb5_v, [ef_v[pl.ds(TPW, TPW)]])
    x = (s1 + b1) * w_v[pl.ds(0, TPW)] + (s2 + b2) * w_v[pl.ds(TPW, TPW)]
    out_v[...] = 1.0 / (1.0 + jnp.exp(-x))
    pltpu.sync_copy(out_v, out_hbm.at[pl.ds(base, TPW)])


def _sc_combine(op, pos, ef, wts, eb5):
    mesh = plsc.VectorSubcoreMesh(core_axis_name="c", subcore_axis_name="s")
    return pl.kernel(
        _sc_combine_body,
        out_type=jax.ShapeDtypeStruct((B,), jnp.float32),
        mesh=mesh,
        scratch_types=[
            pltpu.VMEM((CAP,), jnp.float32),
            pltpu.VMEM((2 * TPW,), jnp.int32),
            pltpu.VMEM((2 * TPW,), jnp.int32),
            pltpu.VMEM((2 * TPW,), jnp.float32),
            pltpu.VMEM((E,), jnp.float32),
            pltpu.VMEM((TPW,), jnp.float32),
        ],
        compiler_params=pltpu.CompilerParams(needs_layout_passes=False),
    )(op, pos, ef, wts[:, 0], wts[:, 1], eb5)


def _expert_body(be_ref, act_ref, z_ref, disp_ref, w1_ref, b1_ref, w2_ref,
                 b2_ref, w3_ref, b3_ref, w4_ref, b4_ref, w5_ref, out_ref):
    j = pl.program_id(0)

    @pl.when(act_ref[j] != 0)
    def _active():
        dval = disp_ref[0]
        io_b = lax.broadcasted_iota(jnp.int32, (B, R), 0)
        oh = (io_b == dval).astype(jnp.float32)
        x = lax.dot_general(oh, z_ref[...], (((0,), (0,)), ((), ())),
                            preferred_element_type=jnp.float32)
        h = _gelu(_dott(x, w1_ref[0]) + b1_ref[0])
        h = _gelu(_dott(h, w2_ref[0]) + b2_ref[0])
        h = _gelu(_dott(h, w3_ref[0]) + b3_ref[0])
        h = _gelu(_dott(h, w4_ref[0]) + b4_ref[0])
        out_ref[...] = jnp.sum(h * w5_ref[0], axis=1, keepdims=True)

    @pl.when(act_ref[j] == 0)
    def _inactive():
        out_ref[...] = jnp.zeros((R, 1), jnp.float32)


def _expert_call(bexp, bact, z, disp, eW1, eb1, eW2, eb2, eW3, eb3, eW4, eb4,
                 eW5):
    def _wmap(j, be, act):
        return (be[j], 0, 0)

    grid_spec = pltpu.PrefetchScalarGridSpec(
        num_scalar_prefetch=2,
        grid=(NBLK,),
        in_specs=[
            pl.BlockSpec((B, D), lambda j, be, act: (0, 0)),
            pl.BlockSpec((1, 1, R), lambda j, be, act: (j, 0, 0)),
            pl.BlockSpec((1, 1024, 1024), _wmap),
            pl.BlockSpec((1, 1, 1024), _wmap),
            pl.BlockSpec((1, 512, 1024), _wmap),
            pl.BlockSpec((1, 1, 512), _wmap),
            pl.BlockSpec((1, 256, 512), _wmap),
            pl.BlockSpec((1, 1, 256), _wmap),
            pl.BlockSpec((1, 128, 256), _wmap),
            pl.BlockSpec((1, 1, 128), _wmap),
            pl.BlockSpec((1, 1, 128), _wmap),
        ],
        out_specs=pl.BlockSpec((R, 1), lambda j, be, act: (j, 0)),
    )
    return pl.pallas_call(
        _expert_body,
        grid_spec=grid_spec,
        out_shape=jax.ShapeDtypeStruct((CAP, 1), jnp.float32),
    )(bexp, bact, z, disp.reshape(NBLK, 1, R),
      eW1, eb1.reshape(E, 1, 1024), eW2, eb2.reshape(E, 1, 512),
      eW3, eb3.reshape(E, 1, 256), eW4, eb4.reshape(E, 1, 128), eW5)


def kernel(con_output, proj_W, proj_b, in_proj_W, in_proj_b, out_proj_W,
           out_proj_b, ln_gamma, ln_beta, gate_W, gate_b,
           eW1, eb1, eW2, eb2, eW3, eb3, eW4, eb4, eW5, eb5):
    qkv = _qkv_call(con_output, proj_W, proj_b, in_proj_W, in_proj_b)
    o = _attn_call(qkv)
    z, disp, bexp, bact, pos, ef, wts = _route_call(
        o, out_proj_W, out_proj_b, ln_gamma, ln_beta, gate_W, gate_b)
    op = _expert_call(bexp.reshape(NBLK), bact.reshape(NBLK), z, disp,
                      eW1, eb1, eW2, eb2, eW3, eb3, eW4, eb4, eW5)
    return _sc_combine(op.reshape(CAP), pos.reshape(P), ef.reshape(P), wts,
                       eb5.reshape(E))


# speedup vs baseline: 2.1838x; 1.1437x over previous
import functools

import jax
import jax.numpy as jnp
import numpy as np
from jax import lax
from jax.experimental import pallas as pl
from jax.experimental.pallas import tpu as pltpu
from jax.experimental.pallas import tpu_sc as plsc

B = 512
D = 1024
NH = 8
DH = D // NH
E = 16
K = 2
P = B * K
R = 128
CAP = 3072
NBLK = CAP // R
SC_NC = 2
SC_NS = 16
NW = SC_NC * SC_NS


def _dott(a, b):
    return lax.dot_general(a, b, (((1,), (1,)), ((), ())),
                           preferred_element_type=jnp.float32)


def _dot(a, b):
    return lax.dot_general(a, b, (((1,), (0,)), ((), ())),
                           preferred_element_type=jnp.float32)


def _gelu(x):
    return 0.5 * x * (1.0 + lax.erf(x * np.float32(1.0 / np.sqrt(2.0))))


def _qkv_body(x_ref, pw_ref, pb_ref, iw_ref, ib_ref, qkv_ref):
    proj = _dott(x_ref[...], pw_ref[...]) + pb_ref[...]
    qkv_ref[...] = _dott(proj, iw_ref[...]) + ib_ref[...]


def _qkv_call(x, proj_W, proj_b, in_proj_W, in_proj_b):
    return pl.pallas_call(
        _qkv_body,
        out_shape=jax.ShapeDtypeStruct((B, 3 * D), jnp.float32),
    )(x, proj_W, proj_b.reshape(1, D), in_proj_W, in_proj_b.reshape(1, 3 * D))


def _attn_body(q_ref, k_ref, v_ref, o_ref):
    s = _dott(q_ref[...], k_ref[...]) * np.float32(1.0 / np.sqrt(DH))
    m = jnp.max(s, axis=-1, keepdims=True)
    e = jnp.exp(s - m)
    attn = e / jnp.sum(e, axis=-1, keepdims=True)
    o_ref[...] = _dot(attn, v_ref[...])


def _attn_call(qkv):
    return pl.pallas_call(
        _attn_body,
        grid=(NH,),
        in_specs=[
            pl.BlockSpec((B, DH), lambda h: (0, h)),
            pl.BlockSpec((B, DH), lambda h: (0, NH + h)),
            pl.BlockSpec((B, DH), lambda h: (0, 2 * NH + h)),
        ],
        out_specs=pl.BlockSpec((B, DH), lambda h: (0, h)),
        out_shape=jax.ShapeDtypeStruct((B, D), jnp.float32),
    )(qkv, qkv, qkv)


def _route_body(o_ref, ow_ref, ob_ref, lg_ref, lb_ref, gw_ref, gb_ref,
                z_ref, disp_ref, bexp_ref, bact_ref, pos_ref, ef_ref, wts_ref):
    ao = _dott(o_ref[...], ow_ref[...]) + ob_ref[...]
    mu = jnp.mean(ao, axis=-1, keepdims=True)
    var = jnp.mean((ao - mu) ** 2, axis=-1, keepdims=True)
    z = (ao - mu) / jnp.sqrt(var + np.float32(1e-5)) * lg_ref[...] + lb_ref[...]
    z_ref[...] = z

    g = _dott(z, gw_ref[...]) + gb_ref[...]
    ioL = lax.broadcasted_iota(jnp.int32, (B, E), 1)
    l1 = jnp.max(g, axis=1, keepdims=True)
    i1 = jnp.min(jnp.where(g == l1, ioL, E), axis=1, keepdims=True)
    gm = jnp.where(ioL == i1, -jnp.inf, g)
    l2 = jnp.max(gm, axis=1, keepdims=True)
    i2 = jnp.min(jnp.where(gm == l2, ioL, E), axis=1, keepdims=True)
    w1 = 1.0 / (1.0 + jnp.exp(l2 - l1))
    wts_ref[...] = jnp.concatenate([w1, 1.0 - w1], axis=1)
    ef_ref[...] = jnp.concatenate([i1, i2], axis=0)

    oh = jnp.concatenate([ioL == i1, ioL == i2], axis=0).astype(jnp.float32)
    counts = jnp.sum(oh, axis=0, keepdims=True).astype(jnp.int32)
    padded = ((counts + (R - 1)) // R) * R
    mr = lax.broadcasted_iota(jnp.int32, (E, E), 0)
    mc = lax.broadcasted_iota(jnp.int32, (E, E), 1)
    mstrict = (mr < mc).astype(jnp.float32)
    off_row = _dot(padded.astype(jnp.float32), mstrict)
    tr = lax.broadcasted_iota(jnp.int32, (P, P), 0)
    tc = lax.broadcasted_iota(jnp.int32, (P, P), 1)
    tstrict = (tc < tr).astype(jnp.float32)
    rankmat = _dot(tstrict, oh)
    rank = jnp.sum(rankmat * oh, axis=1, keepdims=True)
    posoff = jnp.sum(oh * off_row, axis=1, keepdims=True)
    pos = (posoff + rank).astype(jnp.int32)
    pos_ref[...] = pos

    pos_oh = (lax.broadcasted_iota(jnp.int32, (P, CAP), 1) == pos)
    tokf = (lax.broadcasted_iota(jnp.int32, (1, P), 1) % B).astype(jnp.float32)
    disp_ref[...] = _dot(tokf, pos_oh.astype(jnp.float32)).astype(jnp.int32)

    bs = lax.broadcasted_iota(jnp.int32, (NBLK, 1), 0) * R
    offi = off_row.astype(jnp.int32)
    bexp_ref[...] = jnp.sum((offi <= bs).astype(jnp.int32), axis=1,
                            keepdims=True) - 1
    total = jnp.sum(padded)
    bact_ref[...] = (bs < total).astype(jnp.int32)


def _route_call(o, out_proj_W, out_proj_b, ln_gamma, ln_beta, gate_W, gate_b):
    return pl.pallas_call(
        _route_body,
        out_shape=(
            jax.ShapeDtypeStruct((B, D), jnp.float32),
            jax.ShapeDtypeStruct((1, CAP), jnp.int32),
            jax.ShapeDtypeStruct((NBLK, 1), jnp.int32),
            jax.ShapeDtypeStruct((NBLK, 1), jnp.int32),
            jax.ShapeDtypeStruct((P, 1), jnp.int32),
            jax.ShapeDtypeStruct((P, 1), jnp.int32),
            jax.ShapeDtypeStruct((B, K), jnp.float32),
        ),
    )(o, out_proj_W, out_proj_b.reshape(1, D), ln_gamma.reshape(1, D),
      ln_beta.reshape(1, D), gate_W, gate_b.reshape(1, E))


TPW = B // NW


def _sc_combine_body(op_hbm, pos_hbm, ef_hbm, w1_hbm, w2_hbm, eb5_hbm,
                     out_hbm, op_v, pos_v, ef_v, w_v, eb5_v, out_v):
    wid = lax.axis_index("s") * SC_NC + lax.axis_index("c")
    base = wid * TPW
    pltpu.sync_copy(op_hbm, op_v)
    pltpu.sync_copy(pos_hbm.at[pl.ds(base, TPW)], pos_v.at[pl.ds(0, TPW)])
    pltpu.sync_copy(pos_hbm.at[pl.ds(B + base, TPW)], pos_v.at[pl.ds(TPW, TPW)])
    pltpu.sync_copy(ef_hbm.at[pl.ds(base, TPW)], ef_v.at[pl.ds(0, TPW)])
    pltpu.sync_copy(ef_hbm.at[pl.ds(B + base, TPW)], ef_v.at[pl.ds(TPW, TPW)])
    pltpu.sync_copy(w1_hbm.at[pl.ds(base, TPW)], w_v.at[pl.ds(0, TPW)])
    pltpu.sync_copy(w2_hbm.at[pl.ds(base, TPW)], w_v.at[pl.ds(TPW, TPW)])
    pltpu.sync_copy(eb5_hbm, eb5_v)
    s1 = plsc.load_gather(op_v, [pos_v[pl.ds(0, TPW)]])
    s2 = plsc.load_gather(op_v, [pos_v[pl.ds(TPW, TPW)]])
    b1 = plsc.load_gather(eb5_v, [ef_v[pl.ds(0, TPW)]])
    b2 = plsc.load_gather(eb5_v, [ef_v[pl.ds(TPW, TPW)]])
    x = (s1 + b1) * w_v[pl.ds(0, TPW)] + (s2 + b2) * w_v[pl.ds(TPW, TPW)]
    out_v[...] = 1.0 / (1.0 + jnp.exp(-x))
    pltpu.sync_copy(out_v, out_hbm.at[pl.ds(base, TPW)])


def _sc_combine(op, pos, ef, wts, eb5):
    mesh = plsc.VectorSubcoreMesh(core_axis_name="c", subcore_axis_name="s")
    return pl.kernel(
        _sc_combine_body,
        out_type=jax.ShapeDtypeStruct((B,), jnp.float32),
        mesh=mesh,
        scratch_types=[
            pltpu.VMEM((CAP,), jnp.float32),
            pltpu.VMEM((2 * TPW,), jnp.int32),
            pltpu.VMEM((2 * TPW,), jnp.int32),
            pltpu.VMEM((2 * TPW,), jnp.float32),
            pltpu.VMEM((E,), jnp.float32),
            pltpu.VMEM((TPW,), jnp.float32),
        ],
        compiler_params=pltpu.CompilerParams(needs_layout_passes=False),
    )(op, pos, ef, wts[:, 0], wts[:, 1], eb5)


def _expert_body(be_ref, act_ref, z_ref, disp_ref, w1_ref, b1_ref, w2_ref,
                 b2_ref, w3_ref, b3_ref, w4_ref, b4_ref, w5_ref, out_ref):
    j = pl.program_id(0)

    @pl.when(act_ref[j] != 0)
    def _active():
        dval = disp_ref[0]
        io_b = lax.broadcasted_iota(jnp.int32, (B, R), 0)
        oh = (io_b == dval).astype(jnp.float32)
        x = lax.dot_general(oh, z_ref[...], (((0,), (0,)), ((), ())),
                            preferred_element_type=jnp.float32)
        h = _gelu(_dott(x, w1_ref[0]) + b1_ref[0])
        h = _gelu(_dott(h, w2_ref[0]) + b2_ref[0])
        h = _gelu(_dott(h, w3_ref[0]) + b3_ref[0])
        h = _gelu(_dott(h, w4_ref[0]) + b4_ref[0])
        out_ref[...] = jnp.sum(h * w5_ref[0], axis=1, keepdims=True)

    @pl.when(act_ref[j] == 0)
    def _inactive():
        out_ref[...] = jnp.zeros((R, 1), jnp.float32)


def _expert_call(bexp, bact, z, disp, eW1, eb1, eW2, eb2, eW3, eb3, eW4, eb4,
                 eW5):
    def _wmap(j, be, act):
        return (be[j], 0, 0)

    grid_spec = pltpu.PrefetchScalarGridSpec(
        num_scalar_prefetch=2,
        grid=(NBLK,),
        in_specs=[
            pl.BlockSpec((B, D), lambda j, be, act: (0, 0)),
            pl.BlockSpec((1, 1, R), lambda j, be, act: (j, 0, 0)),
            pl.BlockSpec((1, 1024, 1024), _wmap),
            pl.BlockSpec((1, 1, 1024), _wmap),
            pl.BlockSpec((1, 512, 1024), _wmap),
            pl.BlockSpec((1, 1, 512), _wmap),
            pl.BlockSpec((1, 256, 512), _wmap),
            pl.BlockSpec((1, 1, 256), _wmap),
            pl.BlockSpec((1, 128, 256), _wmap),
            pl.BlockSpec((1, 1, 128), _wmap),
            pl.BlockSpec((1, 1, 128), _wmap),
        ],
        out_specs=pl.BlockSpec((R, 1), lambda j, be, act: (j, 0)),
    )
    return pl.pallas_call(
        _expert_body,
        grid_spec=grid_spec,
        out_shape=jax.ShapeDtypeStruct((CAP, 1), jnp.float32),
    )(bexp, bact, z, disp.reshape(NBLK, 1, R),
      eW1, eb1.reshape(E, 1, 1024), eW2, eb2.reshape(E, 1, 512),
      eW3, eb3.reshape(E, 1, 256), eW4, eb4.reshape(E, 1, 128), eW5)


def kernel(con_output, proj_W, proj_b, in_proj_W, in_proj_b, out_proj_W,
           out_proj_b, ln_gamma, ln_beta, gate_W, gate_b,
           eW1, eb1, eW2, eb2, eW3, eb3, eW4, eb4, eW5, eb5):
    qkv = _qkv_call(con_output, proj_W, proj_b, in_proj_W, in_proj_b)
    o = _attn_call(qkv)
    z, disp, bexp, bact, pos, ef, wts = _route_call(
        o, out_proj_W, out_proj_b, ln_gamma, ln_beta, gate_W, gate_b)
    op = _expert_call(bexp.reshape(NBLK), bact.reshape(NBLK), z, disp,
                      eW1, eb1, eW2, eb2, eW3, eb3, eW4, eb4, eW5)
    return _sc_combine(op.reshape(CAP), pos.reshape(P), ef.reshape(P), wts,
                       eb5.reshape(E))


# speedup vs baseline: 2.4006x; 1.0993x over previous
import functools

import jax
import jax.numpy as jnp
import numpy as np
from jax import lax
from jax.experimental import pallas as pl
from jax.experimental.pallas import tpu as pltpu
from jax.experimental.pallas import tpu_sc as plsc

B = 512
D = 1024
NH = 8
DH = D // NH
E = 16
K = 2
P = B * K
R = 128
CAP = 3072
NBLK = CAP // R
SC_NC = 2
SC_NS = 16
NW = SC_NC * SC_NS


def _dott(a, b):
    return lax.dot_general(a, b, (((1,), (1,)), ((), ())),
                           preferred_element_type=jnp.float32)


def _dot(a, b):
    return lax.dot_general(a, b, (((1,), (0,)), ((), ())),
                           preferred_element_type=jnp.float32)


def _gelu(x):
    return 0.5 * x * (1.0 + lax.erf(x * np.float32(1.0 / np.sqrt(2.0))))


def _front_body(x_ref, pw_ref, pb_ref, iw_ref, ib_ref, ow_ref, ob_ref,
                lg_ref, lb_ref, gw_ref, gb_ref,
                z_ref, disp_ref, bexp_ref, bact_ref, pos_ref, ef_ref, wts_ref):
    proj = _dott(x_ref[...], pw_ref[...]) + pb_ref[...]
    qkv = _dott(proj, iw_ref[...]) + ib_ref[...]
    heads = []
    for h in range(NH):
        q = qkv[:, h * DH:(h + 1) * DH]
        k = qkv[:, D + h * DH:D + (h + 1) * DH]
        v = qkv[:, 2 * D + h * DH:2 * D + (h + 1) * DH]
        s = _dott(q, k) * np.float32(1.0 / np.sqrt(DH))
        m = jnp.max(s, axis=-1, keepdims=True)
        e = jnp.exp(s - m)
        attn = e / jnp.sum(e, axis=-1, keepdims=True)
        heads.append(_dot(attn, v))
    o = jnp.concatenate(heads, axis=1)
    ao = _dott(o, ow_ref[...]) + ob_ref[...]
    mu = jnp.mean(ao, axis=-1, keepdims=True)
    var = jnp.mean((ao - mu) ** 2, axis=-1, keepdims=True)
    z = (ao - mu) / jnp.sqrt(var + np.float32(1e-5)) * lg_ref[...] + lb_ref[...]
    z_ref[...] = z

    g = _dott(z, gw_ref[...]) + gb_ref[...]
    ioL = lax.broadcasted_iota(jnp.int32, (B, E), 1)
    l1 = jnp.max(g, axis=1, keepdims=True)
    i1 = jnp.min(jnp.where(g == l1, ioL, E), axis=1, keepdims=True)
    gm = jnp.where(ioL == i1, -jnp.inf, g)
    l2 = jnp.max(gm, axis=1, keepdims=True)
    i2 = jnp.min(jnp.where(gm == l2, ioL, E), axis=1, keepdims=True)
    w1 = 1.0 / (1.0 + jnp.exp(l2 - l1))
    wts_ref[...] = jnp.concatenate([w1, 1.0 - w1], axis=1)
    ef_ref[...] = jnp.concatenate([i1, i2], axis=0)

    oh = jnp.concatenate([ioL == i1, ioL == i2], axis=0).astype(jnp.float32)
    counts = jnp.sum(oh, axis=0, keepdims=True).astype(jnp.int32)
    padded = ((counts + (R - 1)) // R) * R
    mr = lax.broadcasted_iota(jnp.int32, (E, E), 0)
    mc = lax.broadcasted_iota(jnp.int32, (E, E), 1)
    mstrict = (mr < mc).astype(jnp.float32)
    off_row = _dot(padded.astype(jnp.float32), mstrict)
    tr = lax.broadcasted_iota(jnp.int32, (P, P), 0)
    tc = lax.broadcasted_iota(jnp.int32, (P, P), 1)
    tstrict = (tc < tr).astype(jnp.float32)
    rankmat = _dot(tstrict, oh)
    rank = jnp.sum(rankmat * oh, axis=1, keepdims=True)
    posoff = jnp.sum(oh * off_row, axis=1, keepdims=True)
    pos = (posoff + rank).astype(jnp.int32)
    pos_ref[...] = pos

    pos_oh = (lax.broadcasted_iota(jnp.int32, (P, CAP), 1) == pos)
    tokf = (lax.broadcasted_iota(jnp.int32, (1, P), 1) % B).astype(jnp.float32)
    disp_ref[...] = _dot(tokf, pos_oh.astype(jnp.float32)).astype(jnp.int32)

    bs = lax.broadcasted_iota(jnp.int32, (NBLK, 1), 0) * R
    offi = off_row.astype(jnp.int32)
    bexp_ref[...] = jnp.sum((offi <= bs).astype(jnp.int32), axis=1,
                            keepdims=True) - 1
    total = jnp.sum(padded)
    bact_ref[...] = (bs < total).astype(jnp.int32)


def _front_call(x, proj_W, proj_b, in_proj_W, in_proj_b, out_proj_W,
                out_proj_b, ln_gamma, ln_beta, gate_W, gate_b):
    return pl.pallas_call(
        _front_body,
        out_shape=(
            jax.ShapeDtypeStruct((B, D), jnp.float32),
            jax.ShapeDtypeStruct((1, CAP), jnp.int32),
            jax.ShapeDtypeStruct((NBLK, 1), jnp.int32),
            jax.ShapeDtypeStruct((NBLK, 1), jnp.int32),
            jax.ShapeDtypeStruct((P, 1), jnp.int32),
            jax.ShapeDtypeStruct((P, 1), jnp.int32),
            jax.ShapeDtypeStruct((B, K), jnp.float32),
        ),
    )(x, proj_W, proj_b.reshape(1, D), in_proj_W, in_proj_b.reshape(1, 3 * D),
      out_proj_W, out_proj_b.reshape(1, D), ln_gamma.reshape(1, D),
      ln_beta.reshape(1, D), gate_W, gate_b.reshape(1, E))


TPW = B // NW


def _sc_combine_body(op_hbm, pos_hbm, ef_hbm, w1_hbm, w2_hbm, eb5_hbm,
                     out_hbm, op_v, pos_v, ef_v, w_v, eb5_v, out_v):
    wid = lax.axis_index("s") * SC_NC + lax.axis_index("c")
    base = wid * TPW
    pltpu.sync_copy(op_hbm, op_v)
    pltpu.sync_copy(pos_hbm.at[pl.ds(base, TPW)], pos_v.at[pl.ds(0, TPW)])
    pltpu.sync_copy(pos_hbm.at[pl.ds(B + base, TPW)], pos_v.at[pl.ds(TPW, TPW)])
    pltpu.sync_copy(ef_hbm.at[pl.ds(base, TPW)], ef_v.at[pl.ds(0, TPW)])
    pltpu.sync_copy(ef_hbm.at[pl.ds(B + base, TPW)], ef_v.at[pl.ds(TPW, TPW)])
    pltpu.sync_copy(w1_hbm.at[pl.ds(base, TPW)], w_v.at[pl.ds(0, TPW)])
    pltpu.sync_copy(w2_hbm.at[pl.ds(base, TPW)], w_v.at[pl.ds(TPW, TPW)])
    pltpu.sync_copy(eb5_hbm, eb5_v)
    s1 = plsc.load_gather(op_v, [pos_v[pl.ds(0, TPW)]])
    s2 = plsc.load_gather(op_v, [pos_v[pl.ds(TPW, TPW)]])
    b1 = plsc.load_gather(eb5_v, [ef_v[pl.ds(0, TPW)]])
    b2 = plsc.load_gather(eb5_v, [ef_v[pl.ds(TPW, TPW)]])
    x = (s1 + b1) * w_v[pl.ds(0, TPW)] + (s2 + b2) * w_v[pl.ds(TPW, TPW)]
    out_v[...] = 1.0 / (1.0 + jnp.exp(-x))
    pltpu.sync_copy(out_v, out_hbm.at[pl.ds(base, TPW)])


def _sc_combine(op, pos, ef, wts, eb5):
    mesh = plsc.VectorSubcoreMesh(core_axis_name="c", subcore_axis_name="s")
    return pl.kernel(
        _sc_combine_body,
        out_type=jax.ShapeDtypeStruct((B,), jnp.float32),
        mesh=mesh,
        scratch_types=[
            pltpu.VMEM((CAP,), jnp.float32),
            pltpu.VMEM((2 * TPW,), jnp.int32),
            pltpu.VMEM((2 * TPW,), jnp.int32),
            pltpu.VMEM((2 * TPW,), jnp.float32),
            pltpu.VMEM((E,), jnp.float32),
            pltpu.VMEM((TPW,), jnp.float32),
        ],
        compiler_params=pltpu.CompilerParams(needs_layout_passes=False),
    )(op, pos, ef, wts[:, 0], wts[:, 1], eb5)


def _expert_body(be_ref, act_ref, z_ref, disp_ref, w1_ref, b1_ref, w2_ref,
                 b2_ref, w3_ref, b3_ref, w4_ref, b4_ref, w5_ref, out_ref):
    j = pl.program_id(0)

    @pl.when(act_ref[j] != 0)
    def _active():
        dval = disp_ref[0]
        io_b = lax.broadcasted_iota(jnp.int32, (B, R), 0)
        oh = (io_b == dval).astype(jnp.float32)
        x = lax.dot_general(oh, z_ref[...], (((0,), (0,)), ((), ())),
                            preferred_element_type=jnp.float32)
        h = _gelu(_dott(x, w1_ref[0]) + b1_ref[0])
        h = _gelu(_dott(h, w2_ref[0]) + b2_ref[0])
        h = _gelu(_dott(h, w3_ref[0]) + b3_ref[0])
        h = _gelu(_dott(h, w4_ref[0]) + b4_ref[0])
        out_ref[...] = jnp.sum(h * w5_ref[0], axis=1, keepdims=True)

    @pl.when(act_ref[j] == 0)
    def _inactive():
        out_ref[...] = jnp.zeros((R, 1), jnp.float32)


def _expert_call(bexp, bact, z, disp, eW1, eb1, eW2, eb2, eW3, eb3, eW4, eb4,
                 eW5):
    def _wmap(j, be, act):
        return (be[j], 0, 0)

    grid_spec = pltpu.PrefetchScalarGridSpec(
        num_scalar_prefetch=2,
        grid=(NBLK,),
        in_specs=[
            pl.BlockSpec((B, D), lambda j, be, act: (0, 0)),
            pl.BlockSpec((1, 1, R), lambda j, be, act: (j, 0, 0)),
            pl.BlockSpec((1, 1024, 1024), _wmap),
            pl.BlockSpec((1, 1, 1024), _wmap),
            pl.BlockSpec((1, 512, 1024), _wmap),
            pl.BlockSpec((1, 1, 512), _wmap),
            pl.BlockSpec((1, 256, 512), _wmap),
            pl.BlockSpec((1, 1, 256), _wmap),
            pl.BlockSpec((1, 128, 256), _wmap),
            pl.BlockSpec((1, 1, 128), _wmap),
            pl.BlockSpec((1, 1, 128), _wmap),
        ],
        out_specs=pl.BlockSpec((R, 1), lambda j, be, act: (j, 0)),
    )
    return pl.pallas_call(
        _expert_body,
        grid_spec=grid_spec,
        out_shape=jax.ShapeDtypeStruct((CAP, 1), jnp.float32),
    )(bexp, bact, z, disp.reshape(NBLK, 1, R),
      eW1, eb1.reshape(E, 1, 1024), eW2, eb2.reshape(E, 1, 512),
      eW3, eb3.reshape(E, 1, 256), eW4, eb4.reshape(E, 1, 128), eW5)


def kernel(con_output, proj_W, proj_b, in_proj_W, in_proj_b, out_proj_W,
           out_proj_b, ln_gamma, ln_beta, gate_W, gate_b,
           eW1, eb1, eW2, eb2, eW3, eb3, eW4, eb4, eW5, eb5):
    z, disp, bexp, bact, pos, ef, wts = _front_call(
        con_output, proj_W, proj_b, in_proj_W, in_proj_b, out_proj_W,
        out_proj_b, ln_gamma, ln_beta, gate_W, gate_b)
    op = _expert_call(bexp.reshape(NBLK), bact.reshape(NBLK), z, disp,
                      eW1, eb1, eW2, eb2, eW3, eb3, eW4, eb4, eW5)
    return _sc_combine(op.reshape(CAP), pos.reshape(P), ef.reshape(P), wts,
                       eb5.reshape(E))
